# baseline (speedup 1.0000x reference)
import jax
import jax.numpy as jnp
from jax import lax
from jax.experimental import pallas as pl
from jax.experimental.pallas import tpu as pltpu
from jax.experimental.pallas import tpu_sc as plsc

NUM_STATES = 8192
BATCH = 16
SEQLEN = 2048
HALF = SEQLEN // 2
NG = 8
GW = HALF // NG
NTILES = 2 * BATCH


def _sc_gather_body(data_hbm, tp_hbm, ip_hbm, out_hbm, row_v, idx_v, vals_v,
                    idx1_v, val1_v, sem):
    c = lax.axis_index("c")
    s = lax.axis_index("s")
    wid = s * 2 + c
    b = wid // 2
    half = wid % 2
    base = half * HALF

    pltpu.sync_copy(data_hbm.at[pl.ds(b * SEQLEN, SEQLEN)],
                    row_v.at[pl.ds(0, SEQLEN)])

    iota = lax.iota(jnp.int32, 16)
    for g in range(NG):
        for u in range(GW // 16):
            j = g * (GW // 16) + u
            pos = base + j * 16
            prev = row_v[pl.ds(pos, 16)]
            nxt = row_v[pl.ds(pos + 1, 16)]
            idx_v[g, pl.ds(u * 16, 16)] = jnp.clip(
                prev * NUM_STATES + nxt, 0, NUM_STATES * NUM_STATES - 1)

    cps = [pltpu.async_copy(tp_hbm.at[idx_v.at[g]], vals_v.at[g], sem)
           for g in range(NG)]
    for cp in cps:
        cp.wait()

    @pl.when(half == 1)
    def _():
        head = row_v[pl.ds(0, 16)]
        idx1_v[...] = jnp.broadcast_to(head[0], (16,))
        pltpu.async_copy(ip_hbm.at[idx1_v], val1_v, sem).wait()
        tail = vals_v[NG - 1, pl.ds(GW - 16, 16)]
        vals_v[NG - 1, pl.ds(GW - 16, 16)] = jnp.where(
            iota == 15, val1_v[...], tail)

    pltpu.sync_copy(vals_v, out_hbm.at[wid])


def _tc_reduce_body(g_ref, out_ref):
    g = g_ref[...]
    ll = jnp.sum(jnp.log(g), axis=1, keepdims=True)
    m = jnp.max(ll)
    out_ref[...] = jnp.reshape(
        -(m + jnp.log(jnp.sum(jnp.exp(ll - m)))), (1, 1))


def kernel(data, initial_probs, transition_probs):
    data_flat = data.reshape(-1).astype(jnp.int32)
    tp_flat = transition_probs.reshape(-1)

    mesh = plsc.VectorSubcoreMesh(core_axis_name="c", subcore_axis_name="s")
    gathered = pl.kernel(
        _sc_gather_body,
        out_type=jax.ShapeDtypeStruct((NTILES, NG, GW), jnp.float32),
        mesh=mesh,
        scratch_types=[
            pltpu.VMEM((SEQLEN + 16,), jnp.int32),
            pltpu.VMEM((NG, GW), jnp.int32),
            pltpu.VMEM((NG, GW), jnp.float32),
            pltpu.VMEM((16,), jnp.int32),
            pltpu.VMEM((16,), jnp.float32),
            pltpu.SemaphoreType.DMA,
        ],
    )(data_flat, tp_flat, initial_probs)

    terms = gathered.reshape(BATCH, SEQLEN)
    out = pl.pallas_call(
        _tc_reduce_body,
        out_shape=jax.ShapeDtypeStruct((1, 1), jnp.float32),
    )(terms)
    return out.reshape(())


# speedup vs baseline: 1.6995x; 1.6995x over previous
import jax
import jax.numpy as jnp
from jax import lax
from jax.experimental import pallas as pl
from jax.experimental.pallas import tpu as pltpu
from jax.experimental.pallas import tpu_sc as plsc

NUM_STATES = 8192
BATCH = 16
SEQLEN = 2048
HALF = SEQLEN // 2
NBINS = 64
WIN = 32
NFL = 8
NTILES = 2 * BATCH
IGNORED = -1


def _sc_body(data_hbm, tp_hbm, ip_hbm, out_hbm,
             row_v, fib_v, pki_v, pkp_v, cnt_v, start_v, fill_v,
             dml_v, dst_v, outv_v, idx1_v, val1_v,
             cnt_s, start_s, sem, sem1):
    cc = lax.axis_index("c")
    ss = lax.axis_index("s")
    wid = ss * 2 + cc
    b = wid // 2
    half = wid % 2
    base = half * HALF
    iota = lax.iota(jnp.int32, 16)
    zeros16 = jnp.zeros((16,), jnp.int32)

    rk0, _ = plsc.scan_count(zeros16)
    rbase = rk0[0]

    pltpu.sync_copy(data_hbm.at[pl.ds(b * SEQLEN, SEQLEN)],
                    row_v.at[pl.ds(0, SEQLEN)])

    for k in range(NBINS // 16):
        cnt_v[pl.ds(k * 16, 16)] = zeros16

    def p0(j, carry):
        pos = base + j * 16
        prev = jnp.clip(row_v[pl.ds(pos, 16)], 0, NUM_STATES - 1)
        nxt = jnp.clip(row_v[pl.ds(pos + 1, 16)], 0, NUM_STATES - 1)
        fi = (prev << 13) | nxt
        fib_v[pl.ds(j * 16, 16)] = fi
        cb = (fi >> 7) & (NBINS - 1)
        rank, last = plsc.scan_count(cb)
        cur = plsc.load_gather(cnt_v, [cb])
        plsc.store_scatter(cnt_v, [cb], cur + (rank - rbase) + 1, mask=last)
        return carry
    lax.fori_loop(0, HALF // 16, p0, 0)

    carry = jnp.int32(0)
    maxc = jnp.int32(0)
    for k in range(NBINS // 16):
        c16 = cnt_v[pl.ds(k * 16, 16)]
        inc = plsc.cumsum(c16)
        exc = carry + inc - c16
        start_v[pl.ds(k * 16, 16)] = exc
        fill_v[pl.ds(k * 16, 16)] = exc
        carry = carry + inc[15]
        maxc = jnp.maximum(maxc, jnp.max(c16))
        for l in range(16):
            cnt_s[k * 16 + l] = c16[l]
            start_s[k * 16 + l] = exc[l]

    def p2(j, carry):
        fi = fib_v[pl.ds(j * 16, 16)]
        cb = (fi >> 7) & (NBINS - 1)
        rank, last = plsc.scan_count(cb)
        fb = plsc.load_gather(fill_v, [cb])
        slot = jnp.clip(fb + (rank - rbase), 0, HALF + WIN - 1)
        plsc.store_scatter(pki_v, [slot], fi)
        plsc.store_scatter(pkp_v, [slot], j * 16 + iota)
        plsc.store_scatter(fill_v, [cb], slot + 1, mask=last)
        return carry
    lax.fori_loop(0, HALF // 16, p2, 0)

    nrounds = (jnp.clip(maxc, 0, HALF) + WIN - 1) // WIN

    def round_body(r, carry):
        wbase = r * WIN

        def flight_body(f, carry):
            cps = []
            for u in range(NFL):
                cb = f * NFL + u
                st = start_s[cb]
                cn = cnt_s[cb]
                rem = jnp.clip(cn - wbase, 0, WIN)
                segp = jnp.clip(st + wbase, 0, HALF + WIN - 1 - WIN)
                for k in range(WIN // 16):
                    v = pki_v[pl.ds(segp + k * 16, 16)]
                    rows = jnp.clip(v >> 13, 0, NUM_STATES - 1)
                    rows = jnp.where(k * 16 + iota < rem, rows, 0)
                    dml_v[u, pl.ds(k * 16, 16)] = rows
                cps.append(pltpu.async_copy(
                    tp_hbm.at[dml_v.at[u], pl.ds(cb * 128, 128)],
                    dst_v.at[u], sem))
            for cp in cps:
                cp.wait()
            for u in range(NFL):
                cb = f * NFL + u
                st = start_s[cb]
                cn = cnt_s[cb]
                rem = jnp.clip(cn - wbase, 0, WIN)
                segp = jnp.clip(st + wbase, 0, HALF + WIN - 1 - WIN)
                uvec = jnp.full((16,), u, jnp.int32)
                for k in range(WIN // 16):
                    v = pki_v[pl.ds(segp + k * 16, 16)]
                    p = pkp_v[pl.ds(segp + k * 16, 16)]
                    lane = v & 127
                    jvec = k * 16 + iota
                    valid = jvec < rem
                    vals = plsc.load_gather(dst_v, [uvec, jvec, lane])
                    plsc.store_scatter(outv_v, [jnp.clip(p, 0, HALF - 1)],
                                       vals, mask=valid)
            return carry
        return lax.fori_loop(0, NBINS // NFL, flight_body, carry)
    lax.fori_loop(0, nrounds, round_body, 0)

    @pl.when(half == 1)
    def _():
        head = row_v[pl.ds(0, 16)]
        idx1_v[...] = jnp.broadcast_to(head[0], (16,))
        pltpu.async_copy(ip_hbm.at[idx1_v], val1_v, sem1).wait()
        tail = outv_v[pl.ds(HALF - 16, 16)]
        outv_v[pl.ds(HALF - 16, 16)] = jnp.where(
            iota == 15, val1_v[...], tail)

    pltpu.sync_copy(outv_v, out_hbm.at[wid])


def _tc_reduce_body(g_ref, out_ref):
    g = g_ref[...]
    ll = jnp.sum(jnp.log(g), axis=1, keepdims=True)
    m = jnp.max(ll)
    out_ref[...] = jnp.reshape(
        -(m + jnp.log(jnp.sum(jnp.exp(ll - m)))), (1, 1))


def kernel(data, initial_probs, transition_probs):
    data_flat = data.reshape(-1).astype(jnp.int32)

    mesh = plsc.VectorSubcoreMesh(core_axis_name="c", subcore_axis_name="s")
    gathered = pl.kernel(
        _sc_body,
        out_type=jax.ShapeDtypeStruct((NTILES, HALF), jnp.float32),
        mesh=mesh,
        compiler_params=pltpu.CompilerParams(needs_layout_passes=False),
        scratch_types=[
            pltpu.VMEM((SEQLEN + 16,), jnp.int32),
            pltpu.VMEM((HALF + 16,), jnp.int32),
            pltpu.VMEM((HALF + WIN,), jnp.int32),
            pltpu.VMEM((HALF + WIN,), jnp.int32),
            pltpu.VMEM((NBINS,), jnp.int32),
            pltpu.VMEM((NBINS,), jnp.int32),
            pltpu.VMEM((NBINS,), jnp.int32),
            pltpu.VMEM((NFL, WIN), jnp.int32),
            pltpu.VMEM((NFL, WIN, 128), jnp.float32),
            pltpu.VMEM((HALF,), jnp.float32),
            pltpu.VMEM((16,), jnp.int32),
            pltpu.VMEM((16,), jnp.float32),
            pltpu.SMEM((NBINS,), jnp.int32),
            pltpu.SMEM((NBINS,), jnp.int32),
            pltpu.SemaphoreType.DMA,
            pltpu.SemaphoreType.DMA,
        ],
    )(data_flat, transition_probs, initial_probs)

    terms = gathered.reshape(BATCH, SEQLEN)
    out = pl.pallas_call(
        _tc_reduce_body,
        out_shape=jax.ShapeDtypeStruct((1, 1), jnp.float32),
    )(terms)
    return out.reshape(())


# speedup vs baseline: 4.1088x; 2.4177x over previous
import jax
import jax.numpy as jnp
from jax import lax
from jax.experimental import pallas as pl
from jax.experimental.pallas import tpu as pltpu
from jax.experimental.pallas import tpu_sc as plsc

NUM_STATES = 8192
BATCH = 16
SEQLEN = 2048
HALF = SEQLEN // 2
NB = 64
GR = 2
SLOTS = HALF + NB * (GR - 1) + 16
WINR = 384
NW = 3
BUFSZ = NW * WINR + 16
NTILES = 2 * BATCH


def _sc_body(data_hbm, tp_hbm, ip_hbm, out_hbm,
             row_v, fib_v, rows_v, lp_v, cnt_v, start_v, fill_v,
             dst_v, drain_v, outv_v, idx1_v, val1_v, sem, sem1):
    cc = lax.axis_index("c")
    ss = lax.axis_index("s")
    wid = ss * 2 + cc
    b = wid // 2
    half = wid % 2
    base = half * HALF
    iota = lax.iota(jnp.int32, 16)
    zeros16 = jnp.zeros(16, jnp.int32)

    rk0, _ = plsc.scan_count(zeros16)
    rbase = rk0[0]

    pltpu.sync_copy(data_hbm.at[pl.ds(b * SEQLEN, SEQLEN)],
                    row_v.at[pl.ds(0, SEQLEN)])

    def init(j, carry):
        sl = j * 16 + iota
        plsc.store_scatter(rows_v, [sl >> 1, sl & 1], zeros16)
        lp_v[pl.ds(j * 16, 16)] = zeros16 - 1
        return carry
    lax.fori_loop(0, BUFSZ // 16, init, 0)
    for k in range(NB // 16):
        cnt_v[pl.ds(k * 16, 16)] = zeros16

    def p0(j, carry):
        pos = base + j * 16
        prev = jnp.clip(row_v[pl.ds(pos, 16)], 0, NUM_STATES - 1)
        nxt = jnp.clip(row_v[pl.ds(pos + 1, 16)], 0, NUM_STATES - 1)
        fi = (prev << 13) | nxt
        fib_v[pl.ds(j * 16, 16)] = fi
        g = (fi >> 7) & (NB - 1)
        rank, last = plsc.scan_count(g)
        cur = plsc.load_gather(cnt_v, [g])
        plsc.store_scatter(cnt_v, [g], cur + (rank - rbase) + 1, mask=last)
        return carry
    lax.fori_loop(0, HALF // 16, p0, 0)

    carry = jnp.int32(0)
    for k in range(NB // 16):
        c16 = cnt_v[pl.ds(k * 16, 16)]
        pc16 = c16 + (c16 & (GR - 1))
        inc = plsc.cumsum(pc16)
        exc = carry + inc - pc16
        start_v[pl.ds(k * 16, 16)] = exc
        fill_v[pl.ds(k * 16, 16)] = exc
        carry = carry + inc[15]
    used_slots = carry

    def p2(j, carry):
        fi = fib_v[pl.ds(j * 16, 16)]
        g = (fi >> 7) & (NB - 1)
        rank, last = plsc.scan_count(g)
        fb = plsc.load_gather(fill_v, [g])
        slot = jnp.clip(fb + (rank - rbase), 0, SLOTS - 1)
        plsc.store_scatter(rows_v, [slot >> 1, slot & 1], fi >> 13)
        plsc.store_scatter(lp_v, [slot], ((j * 16 + iota) << 7) | (fi & 127))
        plsc.store_scatter(fill_v, [g], slot + 1, mask=last)
        return carry
    lax.fori_loop(0, HALF // 16, p2, 0)

    for win in range(NW):
        win0 = win * WINR

        def fire_group(q, fired):
            c16 = cnt_v[pl.ds(q * 16, 16)]
            s16 = start_v[pl.ds(q * 16, 16)]
            for l in range(16):
                cnt = c16[l]
                stq = s16[l]
                nd = (cnt + (GR - 1)) // GR
                c0 = (q * 16 + l) * 128
                w_lo = jnp.clip((win0 - stq) // GR, 0, nd)
                w_hi = jnp.clip((win0 + WINR - stq) // GR, 0, nd)

                def fire(w, tot):
                    pltpu.async_copy(
                        tp_hbm.at[rows_v.at[(stq >> 1) + w], pl.ds(c0, 128)],
                        dst_v.at[pl.ds(stq + w * GR - win0, GR)], sem)
                    return tot + 1
                fired = lax.fori_loop(w_lo, w_hi, fire, fired)
            return fired
        fired = lax.fori_loop(0, NB // 16, fire_group, jnp.int32(0))

        def drain(w, carry):
            pltpu.make_async_copy(
                out_hbm.at[0, pl.ds(0, GR * 128)], drain_v, sem).wait()
            return carry
        lax.fori_loop(0, fired, drain, 0)

        def extract(j, carry):
            lp = lp_v[pl.ds(win0 + j * 16, 16)]
            valid = lp >= 0
            pos = jnp.clip(lp >> 7, 0, HALF - 1)
            lane = lp & 127
            slot = j * 16 + iota
            vals = plsc.load_gather(dst_v, [slot, lane])
            plsc.store_scatter(outv_v, [pos], vals, mask=valid)
            return carry
        lax.fori_loop(0, WINR // 16, extract, 0)

    @pl.when(half == 1)
    def _():
        head = row_v[pl.ds(0, 16)]
        idx1_v[...] = jnp.broadcast_to(head[0], (16,))
        pltpu.async_copy(ip_hbm.at[idx1_v], val1_v, sem1).wait()
        tail = outv_v[pl.ds(HALF - 16, 16)]
        outv_v[pl.ds(HALF - 16, 16)] = jnp.where(
            iota == 15, val1_v[...], tail)

    pltpu.sync_copy(outv_v, out_hbm.at[wid])


def _tc_reduce_body(g_ref, out_ref):
    g = g_ref[...]
    ll = jnp.sum(jnp.log(g), axis=1, keepdims=True)
    m = jnp.max(ll)
    out_ref[...] = jnp.reshape(
        -(m + jnp.log(jnp.sum(jnp.exp(ll - m)))), (1, 1))


def kernel(data, initial_probs, transition_probs):
    data_flat = data.reshape(-1).astype(jnp.int32)

    mesh = plsc.VectorSubcoreMesh(core_axis_name="c", subcore_axis_name="s")
    gathered = pl.kernel(
        _sc_body,
        out_type=jax.ShapeDtypeStruct((NTILES, HALF), jnp.float32),
        mesh=mesh,
        compiler_params=pltpu.CompilerParams(needs_layout_passes=False),
        scratch_types=[
            pltpu.VMEM((SEQLEN + 16,), jnp.int32),
            pltpu.VMEM((HALF + 16,), jnp.int32),
            pltpu.VMEM((BUFSZ // 2, 2), jnp.int32),
            pltpu.VMEM((BUFSZ,), jnp.int32),
            pltpu.VMEM((NB,), jnp.int32),
            pltpu.VMEM((NB,), jnp.int32),
            pltpu.VMEM((NB,), jnp.int32),
            pltpu.VMEM((WINR, 128), jnp.float32),
            pltpu.VMEM((GR * 128,), jnp.float32),
            pltpu.VMEM((HALF,), jnp.float32),
            pltpu.VMEM((16,), jnp.int32),
            pltpu.VMEM((16,), jnp.float32),
            pltpu.SemaphoreType.DMA,
            pltpu.SemaphoreType.DMA,
        ],
    )(data_flat, transition_probs, initial_probs)

    terms = gathered.reshape(BATCH, SEQLEN)
    out = pl.pallas_call(
        _tc_reduce_body,
        out_shape=jax.ShapeDtypeStruct((1, 1), jnp.float32),
    )(terms)
    return out.reshape(())


# speedup vs baseline: 4.2182x; 1.0266x over previous
import jax
import jax.numpy as jnp
from jax import lax
from jax.experimental import pallas as pl
from jax.experimental.pallas import tpu as pltpu
from jax.experimental.pallas import tpu_sc as plsc

NUM_STATES = 8192
BATCH = 16
SEQLEN = 2048
HALF = SEQLEN // 2
NB = 64
GR = 2
SLOTS = HALF + NB * (GR - 1) + 16
WINR = 384
NW = 3
BUFSZ = NW * WINR + 16
NTILES = 2 * BATCH


def _sc_body(data_hbm, tp_hbm, ip_hbm, out_hbm,
             row_v, fib_v, rows_v, lp_v, cnt_v, start_v, fill_v,
             dst_v, drain_v, outv_v, idx1_v, val1_v, sem, sem1):
    cc = lax.axis_index("c")
    ss = lax.axis_index("s")
    wid = ss * 2 + cc
    b = wid // 2
    half = wid % 2
    base = half * HALF
    iota = lax.iota(jnp.int32, 16)
    zeros16 = jnp.zeros(16, jnp.int32)

    rk0, _ = plsc.scan_count(zeros16)
    rbase = rk0[0]

    pltpu.sync_copy(data_hbm.at[b], row_v.at[pl.ds(0, SEQLEN)])

    def init(j, carry):
        sl = j * 16 + iota
        plsc.store_scatter(rows_v, [sl >> 1, sl & 1], zeros16)
        lp_v[pl.ds(j * 16, 16)] = zeros16 - 1
        return carry
    lax.fori_loop(0, BUFSZ // 16, init, 0)
    for k in range(NB // 16):
        cnt_v[pl.ds(k * 16, 16)] = zeros16

    def p0(j, carry):
        pos = base + j * 16
        prev = jnp.clip(row_v[pl.ds(pos, 16)], 0, NUM_STATES - 1)
        nxt = jnp.clip(row_v[pl.ds(pos + 1, 16)], 0, NUM_STATES - 1)
        fi = (prev << 13) | nxt
        fib_v[pl.ds(j * 16, 16)] = fi
        g = (fi >> 7) & (NB - 1)
        rank, last = plsc.scan_count(g)
        cur = plsc.load_gather(cnt_v, [g])
        plsc.store_scatter(cnt_v, [g], cur + (rank - rbase) + 1, mask=last)
        return carry
    lax.fori_loop(0, HALF // 16, p0, 0)

    carry = jnp.int32(0)
    for k in range(NB // 16):
        c16 = cnt_v[pl.ds(k * 16, 16)]
        pc16 = c16 + (c16 & (GR - 1))
        inc = plsc.cumsum(pc16)
        exc = carry + inc - pc16
        start_v[pl.ds(k * 16, 16)] = exc
        fill_v[pl.ds(k * 16, 16)] = exc
        carry = carry + inc[15]
    used_slots = carry

    def p2(j, carry):
        fi = fib_v[pl.ds(j * 16, 16)]
        g = (fi >> 7) & (NB - 1)
        rank, last = plsc.scan_count(g)
        fb = plsc.load_gather(fill_v, [g])
        slot = jnp.clip(fb + (rank - rbase), 0, SLOTS - 1)
        plsc.store_scatter(rows_v, [slot >> 1, slot & 1], fi >> 13)
        plsc.store_scatter(lp_v, [slot], ((j * 16 + iota) << 7) | (fi & 127))
        plsc.store_scatter(fill_v, [g], slot + 1, mask=last)
        return carry
    lax.fori_loop(0, HALF // 16, p2, 0)

    for win in range(NW):
        win0 = win * WINR

        def fire_group(q, fired):
            c16 = cnt_v[pl.ds(q * 16, 16)]
            s16 = start_v[pl.ds(q * 16, 16)]
            for l in range(16):
                cnt = c16[l]
                stq = s16[l]
                nd = (cnt + (GR - 1)) // GR
                c0 = (q * 16 + l) * 128
                w_lo = jnp.clip((win0 - stq) // GR, 0, nd)
                w_hi = jnp.clip((win0 + WINR - stq) // GR, 0, nd)

                def fire(w, tot):
                    pltpu.async_copy(
                        tp_hbm.at[rows_v.at[(stq >> 1) + w], pl.ds(c0, 128)],
                        dst_v.at[pl.ds(stq + w * GR - win0, GR)], sem)
                    return tot + 1
                fired = lax.fori_loop(w_lo, w_hi, fire, fired)
            return fired
        fired = lax.fori_loop(0, NB // 16, fire_group, jnp.int32(0))

        def drain(w, carry):
            pltpu.make_async_copy(
                out_hbm.at[0, pl.ds(0, GR * 128)], drain_v, sem).wait()
            return carry
        lax.fori_loop(0, fired, drain, 0)

        def extract(j, carry):
            lp = lp_v[pl.ds(win0 + j * 16, 16)]
            valid = lp >= 0
            pos = jnp.clip(lp >> 7, 0, HALF - 1)
            lane = lp & 127
            slot = j * 16 + iota
            vals = plsc.load_gather(dst_v, [slot, lane])
            plsc.store_scatter(outv_v, [pos], vals, mask=valid)
            return carry
        lax.fori_loop(0, WINR // 16, extract, 0)

    @pl.when(half == 1)
    def _():
        head = row_v[pl.ds(0, 16)]
        idx1_v[...] = jnp.broadcast_to(head[0], (16,))
        pltpu.async_copy(ip_hbm.at[idx1_v], val1_v, sem1).wait()
        tail = outv_v[pl.ds(HALF - 16, 16)]
        outv_v[pl.ds(HALF - 16, 16)] = jnp.where(
            iota == 15, val1_v[...], tail)

    pltpu.sync_copy(outv_v, out_hbm.at[b, pl.ds(base, HALF)])


def _tc_reduce_body(g_ref, out_ref):
    g = g_ref[...]
    ll = jnp.sum(jnp.log(g), axis=1, keepdims=True)
    m = jnp.max(ll)
    out_ref[...] = jnp.reshape(
        -(m + jnp.log(jnp.sum(jnp.exp(ll - m)))), (1, 1))


def kernel(data, initial_probs, transition_probs):
    mesh = plsc.VectorSubcoreMesh(core_axis_name="c", subcore_axis_name="s")
    terms = pl.kernel(
        _sc_body,
        out_type=jax.ShapeDtypeStruct((BATCH, SEQLEN), jnp.float32),
        mesh=mesh,
        compiler_params=pltpu.CompilerParams(needs_layout_passes=False),
        scratch_types=[
            pltpu.VMEM((SEQLEN + 16,), jnp.int32),
            pltpu.VMEM((HALF + 16,), jnp.int32),
            pltpu.VMEM((BUFSZ // 2, 2), jnp.int32),
            pltpu.VMEM((BUFSZ,), jnp.int32),
            pltpu.VMEM((NB,), jnp.int32),
            pltpu.VMEM((NB,), jnp.int32),
            pltpu.VMEM((NB,), jnp.int32),
            pltpu.VMEM((WINR, 128), jnp.float32),
            pltpu.VMEM((GR * 128,), jnp.float32),
            pltpu.VMEM((HALF,), jnp.float32),
            pltpu.VMEM((16,), jnp.int32),
            pltpu.VMEM((16,), jnp.float32),
            pltpu.SemaphoreType.DMA,
            pltpu.SemaphoreType.DMA,
        ],
    )(data, transition_probs, initial_probs)

    out = pl.pallas_call(
        _tc_reduce_body,
        out_shape=jax.ShapeDtypeStruct((1, 1), jnp.float32),
    )(terms)
    return out.reshape(())


# speedup vs baseline: 4.3211x; 1.0244x over previous
import jax
import jax.numpy as jnp
from jax import lax
from jax.experimental import pallas as pl
from jax.experimental.pallas import tpu as pltpu
from jax.experimental.pallas import tpu_sc as plsc

NUM_STATES = 8192
BATCH = 16
SEQLEN = 2048
HALF = SEQLEN // 2
NB = 64
GR = 2
SLOTS = HALF + NB * (GR - 1) + 16
WINR = 384
NW = 3
BUFSZ = NW * WINR + 16
NTILES = 2 * BATCH


def _sc_body(data_hbm, tp_hbm, ip_hbm, out_hbm,
             row_v, fib_v, rows_v, lp_v, cnt_v, start_v, fill_v,
             dst_v, drain_v, drain4_v, outv_v, idx1_v, val1_v, sem, sem1):
    cc = lax.axis_index("c")
    ss = lax.axis_index("s")
    wid = ss * 2 + cc
    b = wid // 2
    half = wid % 2
    base = half * HALF
    iota = lax.iota(jnp.int32, 16)
    zeros16 = jnp.zeros(16, jnp.int32)

    rk0, _ = plsc.scan_count(zeros16)
    rbase = rk0[0]

    pltpu.sync_copy(data_hbm.at[b], row_v.at[pl.ds(0, SEQLEN)])

    def init(j, carry):
        sl = j * 16 + iota
        plsc.store_scatter(rows_v, [sl >> 1, sl & 1], zeros16)
        lp_v[pl.ds(j * 16, 16)] = zeros16 - 1
        return carry
    lax.fori_loop(0, BUFSZ // 16, init, 0)
    for k in range(NB // 16):
        cnt_v[pl.ds(k * 16, 16)] = zeros16

    def p0(j, carry):
        pos = base + j * 16
        prev = jnp.clip(row_v[pl.ds(pos, 16)], 0, NUM_STATES - 1)
        nxt = jnp.clip(row_v[pl.ds(pos + 1, 16)], 0, NUM_STATES - 1)
        fi = (prev << 13) | nxt
        fib_v[pl.ds(j * 16, 16)] = fi
        g = (fi >> 7) & (NB - 1)
        rank, last = plsc.scan_count(g)
        cur = plsc.load_gather(cnt_v, [g])
        plsc.store_scatter(cnt_v, [g], cur + (rank - rbase) + 1, mask=last)
        return carry
    lax.fori_loop(0, HALF // 16, p0, 0)

    carry = jnp.int32(0)
    for k in range(NB // 16):
        c16 = cnt_v[pl.ds(k * 16, 16)]
        pc16 = c16 + (c16 & (GR - 1))
        inc = plsc.cumsum(pc16)
        exc = carry + inc - pc16
        start_v[pl.ds(k * 16, 16)] = exc
        fill_v[pl.ds(k * 16, 16)] = exc
        carry = carry + inc[15]
    used_slots = carry

    def p2(j, carry):
        fi = fib_v[pl.ds(j * 16, 16)]
        g = (fi >> 7) & (NB - 1)
        rank, last = plsc.scan_count(g)
        fb = plsc.load_gather(fill_v, [g])
        slot = jnp.clip(fb + (rank - rbase), 0, SLOTS - 1)
        plsc.store_scatter(rows_v, [slot >> 1, slot & 1], fi >> 13)
        plsc.store_scatter(lp_v, [slot], ((j * 16 + iota) << 7) | (fi & 127))
        plsc.store_scatter(fill_v, [g], slot + 1, mask=last)
        return carry
    lax.fori_loop(0, HALF // 16, p2, 0)

    for win in range(NW):
        win0 = win * WINR

        def fire_group(q, fired):
            c16 = cnt_v[pl.ds(q * 16, 16)]
            s16 = start_v[pl.ds(q * 16, 16)]
            for l in range(16):
                cnt = c16[l]
                stq = s16[l]
                nd = (cnt + (GR - 1)) // GR
                c0 = (q * 16 + l) * 128
                w_lo = jnp.clip((win0 - stq) // GR, 0, nd)
                w_hi = jnp.clip((win0 + WINR - stq) // GR, 0, nd)

                def fire(w, tot):
                    pltpu.async_copy(
                        tp_hbm.at[rows_v.at[(stq >> 1) + w], pl.ds(c0, 128)],
                        dst_v.at[pl.ds(stq + w * GR - win0, GR)], sem)
                    return tot + 1
                fired = lax.fori_loop(w_lo, w_hi, fire, fired)
            return fired
        fired = lax.fori_loop(0, NB // 16, fire_group, jnp.int32(0))

        def drain4(w, carry):
            pltpu.make_async_copy(
                out_hbm.at[0, pl.ds(0, 4 * GR * 128)], drain4_v, sem).wait()
            return carry
        lax.fori_loop(0, fired >> 2, drain4, 0)

        def drain1(w, carry):
            pltpu.make_async_copy(
                out_hbm.at[0, pl.ds(0, GR * 128)], drain_v, sem).wait()
            return carry
        lax.fori_loop(0, fired & 3, drain1, 0)

        def extract(j, carry):
            lp = lp_v[pl.ds(win0 + j * 16, 16)]
            valid = lp >= 0
            pos = jnp.clip(lp >> 7, 0, HALF - 1)
            lane = lp & 127
            slot = j * 16 + iota
            vals = plsc.load_gather(dst_v, [slot, lane])
            plsc.store_scatter(outv_v, [pos], vals, mask=valid)
            return carry
        lax.fori_loop(0, WINR // 16, extract, 0)

    @pl.when(half == 1)
    def _():
        head = row_v[pl.ds(0, 16)]
        idx1_v[...] = jnp.broadcast_to(head[0], (16,))
        pltpu.async_copy(ip_hbm.at[idx1_v], val1_v, sem1).wait()
        tail = outv_v[pl.ds(HALF - 16, 16)]
        outv_v[pl.ds(HALF - 16, 16)] = jnp.where(
            iota == 15, val1_v[...], tail)

    pltpu.sync_copy(outv_v, out_hbm.at[b, pl.ds(base, HALF)])


def _tc_reduce_body(g_ref, out_ref):
    g = g_ref[...]
    ll = jnp.sum(jnp.log(g), axis=1, keepdims=True)
    m = jnp.max(ll)
    out_ref[...] = jnp.reshape(
        -(m + jnp.log(jnp.sum(jnp.exp(ll - m)))), (1, 1))


def kernel(data, initial_probs, transition_probs):
    mesh = plsc.VectorSubcoreMesh(core_axis_name="c", subcore_axis_name="s")
    terms = pl.kernel(
        _sc_body,
        out_type=jax.ShapeDtypeStruct((BATCH, SEQLEN), jnp.float32),
        mesh=mesh,
        compiler_params=pltpu.CompilerParams(needs_layout_passes=False),
        scratch_types=[
            pltpu.VMEM((SEQLEN + 16,), jnp.int32),
            pltpu.VMEM((HALF + 16,), jnp.int32),
            pltpu.VMEM((BUFSZ // 2, 2), jnp.int32),
            pltpu.VMEM((BUFSZ,), jnp.int32),
            pltpu.VMEM((NB,), jnp.int32),
            pltpu.VMEM((NB,), jnp.int32),
            pltpu.VMEM((NB,), jnp.int32),
            pltpu.VMEM((WINR, 128), jnp.float32),
            pltpu.VMEM((GR * 128,), jnp.float32),
            pltpu.VMEM((4 * GR * 128,), jnp.float32),
            pltpu.VMEM((HALF,), jnp.float32),
            pltpu.VMEM((16,), jnp.int32),
            pltpu.VMEM((16,), jnp.float32),
            pltpu.SemaphoreType.DMA,
            pltpu.SemaphoreType.DMA,
        ],
    )(data, transition_probs, initial_probs)

    out = pl.pallas_call(
        _tc_reduce_body,
        out_shape=jax.ShapeDtypeStruct((1, 1), jnp.float32),
    )(terms)
    return out.reshape(())


# speedup vs baseline: 4.3290x; 1.0018x over previous
import jax
import jax.numpy as jnp
from jax import lax
from jax.experimental import pallas as pl
from jax.experimental.pallas import tpu as pltpu
from jax.experimental.pallas import tpu_sc as plsc

NUM_STATES = 8192
BATCH = 16
SEQLEN = 2048
HALF = SEQLEN // 2
NB = 64
GR = 2
SLOTS = HALF + NB * (GR - 1) + 16
WINR = 384
NW = 3
BUFSZ = NW * WINR + 16
NTILES = 2 * BATCH


def _sc_body(data_hbm, tp_hbm, ip_hbm, out_hbm,
             row_v, fib_v, rows_v, lp_v, cnt_v, start_v, fill_v,
             dst_v, drain_v, drain8_v, outv_v, idx1_v, val1_v, sem, sem1):
    cc = lax.axis_index("c")
    ss = lax.axis_index("s")
    wid = ss * 2 + cc
    b = wid // 2
    half = wid % 2
    base = half * HALF
    iota = lax.iota(jnp.int32, 16)
    zeros16 = jnp.zeros(16, jnp.int32)

    rk0, _ = plsc.scan_count(zeros16)
    rbase = rk0[0]

    pltpu.sync_copy(data_hbm.at[b], row_v.at[pl.ds(0, SEQLEN)])

    def init(j, carry):
        sl = j * 16 + iota
        plsc.store_scatter(rows_v, [sl >> 1, sl & 1], zeros16)
        lp_v[pl.ds(j * 16, 16)] = zeros16 - 1
        return carry
    lax.fori_loop(0, BUFSZ // 16, init, 0)
    for k in range(NB // 16):
        cnt_v[pl.ds(k * 16, 16)] = zeros16

    def p0(j, carry):
        pos = base + j * 16
        prev = jnp.clip(row_v[pl.ds(pos, 16)], 0, NUM_STATES - 1)
        nxt = jnp.clip(row_v[pl.ds(pos + 1, 16)], 0, NUM_STATES - 1)
        fi = (prev << 13) | nxt
        fib_v[pl.ds(j * 16, 16)] = fi
        g = (fi >> 7) & (NB - 1)
        rank, last = plsc.scan_count(g)
        cur = plsc.load_gather(cnt_v, [g])
        plsc.store_scatter(cnt_v, [g], cur + (rank - rbase) + 1, mask=last)
        return carry
    lax.fori_loop(0, HALF // 16, p0, 0)

    carry = jnp.int32(0)
    for k in range(NB // 16):
        c16 = cnt_v[pl.ds(k * 16, 16)]
        pc16 = c16 + (c16 & (GR - 1))
        inc = plsc.cumsum(pc16)
        exc = carry + inc - pc16
        start_v[pl.ds(k * 16, 16)] = exc
        fill_v[pl.ds(k * 16, 16)] = exc
        carry = carry + inc[15]
    used_slots = carry

    def p2(j, carry):
        fi = fib_v[pl.ds(j * 16, 16)]
        g = (fi >> 7) & (NB - 1)
        rank, last = plsc.scan_count(g)
        fb = plsc.load_gather(fill_v, [g])
        slot = jnp.clip(fb + (rank - rbase), 0, SLOTS - 1)
        plsc.store_scatter(rows_v, [slot >> 1, slot & 1], fi >> 13)
        plsc.store_scatter(lp_v, [slot], ((j * 16 + iota) << 7) | (fi & 127))
        plsc.store_scatter(fill_v, [g], slot + 1, mask=last)
        return carry
    lax.fori_loop(0, HALF // 16, p2, 0)

    for win in range(NW):
        win0 = win * WINR

        def fire_group(q, fired):
            c16 = cnt_v[pl.ds(q * 16, 16)]
            s16 = start_v[pl.ds(q * 16, 16)]
            for l in range(16):
                cnt = c16[l]
                stq = s16[l]
                nd = (cnt + (GR - 1)) // GR
                c0 = (q * 16 + l) * 128
                w_lo = jnp.clip((win0 - stq) // GR, 0, nd)
                w_hi = jnp.clip((win0 + WINR - stq) // GR, 0, nd)

                def fire(w, tot):
                    pltpu.async_copy(
                        tp_hbm.at[rows_v.at[(stq >> 1) + w], pl.ds(c0, 128)],
                        dst_v.at[pl.ds(stq + w * GR - win0, GR)], sem)
                    return tot + 1
                fired = lax.fori_loop(w_lo, w_hi, fire, fired)
            return fired
        fired = lax.fori_loop(0, NB // 16, fire_group, jnp.int32(0))

        def drain8(w, carry):
            pltpu.make_async_copy(
                out_hbm.at[0, pl.ds(0, 8 * GR * 128)], drain8_v, sem).wait()
            return carry
        lax.fori_loop(0, fired >> 3, drain8, 0)

        def drain1(w, carry):
            pltpu.make_async_copy(
                out_hbm.at[0, pl.ds(0, GR * 128)], drain_v, sem).wait()
            return carry
        lax.fori_loop(0, fired & 7, drain1, 0)

        def extract(j, carry):
            lp = lp_v[pl.ds(win0 + j * 16, 16)]
            valid = lp >= 0
            pos = jnp.clip(lp >> 7, 0, HALF - 1)
            lane = lp & 127
            slot = j * 16 + iota
            vals = plsc.load_gather(dst_v, [slot, lane])
            plsc.store_scatter(outv_v, [pos], vals, mask=valid)
            return carry
        lax.fori_loop(0, WINR // 16, extract, 0)

    @pl.when(half == 1)
    def _():
        head = row_v[pl.ds(0, 16)]
        idx1_v[...] = jnp.broadcast_to(head[0], (16,))
        pltpu.async_copy(ip_hbm.at[idx1_v], val1_v, sem1).wait()
        tail = outv_v[pl.ds(HALF - 16, 16)]
        outv_v[pl.ds(HALF - 16, 16)] = jnp.where(
            iota == 15, val1_v[...], tail)

    pltpu.sync_copy(outv_v, out_hbm.at[b, pl.ds(base, HALF)])


def _tc_reduce_body(g_ref, out_ref):
    g = g_ref[...]
    ll = jnp.sum(jnp.log(g), axis=1, keepdims=True)
    m = jnp.max(ll)
    out_ref[...] = jnp.reshape(
        -(m + jnp.log(jnp.sum(jnp.exp(ll - m)))), (1, 1))


def kernel(data, initial_probs, transition_probs):
    mesh = plsc.VectorSubcoreMesh(core_axis_name="c", subcore_axis_name="s")
    terms = pl.kernel(
        _sc_body,
        out_type=jax.ShapeDtypeStruct((BATCH, SEQLEN), jnp.float32),
        mesh=mesh,
        compiler_params=pltpu.CompilerParams(needs_layout_passes=False),
        scratch_types=[
            pltpu.VMEM((SEQLEN + 16,), jnp.int32),
            pltpu.VMEM((HALF + 16,), jnp.int32),
            pltpu.VMEM((BUFSZ // 2, 2), jnp.int32),
            pltpu.VMEM((BUFSZ,), jnp.int32),
            pltpu.VMEM((NB,), jnp.int32),
            pltpu.VMEM((NB,), jnp.int32),
            pltpu.VMEM((NB,), jnp.int32),
            pltpu.VMEM((WINR, 128), jnp.float32),
            pltpu.VMEM((GR * 128,), jnp.float32),
            pltpu.VMEM((8 * GR * 128,), jnp.float32),
            pltpu.VMEM((HALF,), jnp.float32),
            pltpu.VMEM((16,), jnp.int32),
            pltpu.VMEM((16,), jnp.float32),
            pltpu.SemaphoreType.DMA,
            pltpu.SemaphoreType.DMA,
        ],
    )(data, transition_probs, initial_probs)

    out = pl.pallas_call(
        _tc_reduce_body,
        out_shape=jax.ShapeDtypeStruct((1, 1), jnp.float32),
    )(terms)
    return out.reshape(())
